# Initial kernel scaffold; baseline (speedup 1.0000x reference)
import jax
import jax.numpy as jnp
from jax.experimental import pallas as pl


def kernel(class_embedding, entity_embeddings, Wq, bq, Wk, bk, Wv, bv, Wo, bo):
    raise NotImplementedError("write your pallas kernel here")


# speedup vs baseline: 1.2294x; 1.2294x over previous
import jax
import jax.numpy as jnp
from jax import lax
from jax.experimental import pallas as pl
from jax.experimental.pallas import tpu as pltpu

_N = 8192
_D = 768
_H = 256
_K = 32
_NBLK = 8
_BLK = _N // _NBLK

_HI = lax.Precision.HIGHEST


def _logits_body(c_ref, wq_ref, bq_ref, wk_ref, e_ref, out_ref, w_ref):
    @pl.when(pl.program_id(0) == 0)
    def _init():
        q = jnp.dot(c_ref[...], wq_ref[...], preferred_element_type=jnp.float32,
                    precision=_HI) + bq_ref[...]
        w = lax.dot_general(q, wk_ref[...], (((1,), (1,)), ((), ())),
                            preferred_element_type=jnp.float32,
                            precision=_HI)
        w_ref[...] = w

    w = w_ref[...]
    r = lax.dot_general(w, e_ref[...], (((1,), (1,)), ((), ())),
                        preferred_element_type=jnp.float32,
                        precision=_HI)
    out_ref[...] = (r * 0.0625).reshape(_BLK)


def _select_body(l_ref, e_ref, wv_ref, bv_ref, wo_ref, bo_ref, out_ref,
                 rows_ref, sem):
    l = l_ref[...]
    m = jnp.max(l)
    p = jnp.exp(l - m)
    z = jnp.sum(p)
    zinv = 1.0 / z
    flat = (lax.broadcasted_iota(jnp.int32, (_NBLK, _BLK), 0) * _BLK +
            lax.broadcasted_iota(jnp.int32, (_NBLK, _BLK), 1))
    big = jnp.int32(2 ** 30)

    work = p
    idxs = []
    svals = []
    for j in range(_K):
        pj = jnp.max(work)
        ij = jnp.min(jnp.where(work == pj, flat, big))
        idxs.append(ij)
        svals.append(pj * zinv)
        work = jnp.where(flat == ij, 0.0, work)

    for j in range(_K):
        pltpu.make_async_copy(e_ref.at[pl.ds(idxs[j], 1)],
                              rows_ref.at[pl.ds(j, 1)], sem).start()
    for j in range(_K):
        pltpu.make_async_copy(e_ref.at[pl.ds(idxs[j], 1)],
                              rows_ref.at[pl.ds(j, 1)], sem).wait()

    u = rows_ref[0, :][None, :] * svals[0]
    s_sum = svals[0]
    for j in range(1, _K):
        u = u + rows_ref[j, :][None, :] * svals[j]
        s_sum = s_sum + svals[j]

    hv = jnp.dot(u, wv_ref[...], preferred_element_type=jnp.float32,
                 precision=_HI) + s_sum * bv_ref[...]
    out = jnp.dot(hv, wo_ref[...], preferred_element_type=jnp.float32,
                  precision=_HI) + bo_ref[...]
    out_ref[...] = out


def kernel(class_embedding, entity_embeddings, Wq, bq, Wk, bk, Wv, bv, Wo, bo):
    del bk
    c2 = class_embedding.reshape(1, _D)

    logits = pl.pallas_call(
        _logits_body,
        grid=(_NBLK,),
        in_specs=[
            pl.BlockSpec((1, _D), lambda i: (0, 0)),
            pl.BlockSpec((_D, _H), lambda i: (0, 0)),
            pl.BlockSpec((1, _H), lambda i: (0, 0)),
            pl.BlockSpec((_D, _H), lambda i: (0, 0)),
            pl.BlockSpec((_BLK, _D), lambda i: (i, 0)),
        ],
        out_specs=pl.BlockSpec((_BLK,), lambda i: (i,)),
        out_shape=jax.ShapeDtypeStruct((_N,), jnp.float32),
        scratch_shapes=[pltpu.VMEM((1, _D), jnp.float32)],
    )(c2, Wq, bq.reshape(1, _H), Wk, entity_embeddings)
    logits = logits.reshape(_NBLK, _BLK)

    out = pl.pallas_call(
        _select_body,
        in_specs=[
            pl.BlockSpec((_NBLK, _BLK), lambda: (0, 0)),
            pl.BlockSpec(memory_space=pl.ANY),
            pl.BlockSpec((_D, _H), lambda: (0, 0)),
            pl.BlockSpec((1, _H), lambda: (0, 0)),
            pl.BlockSpec((_H, _D), lambda: (0, 0)),
            pl.BlockSpec((1, _D), lambda: (0, 0)),
        ],
        out_specs=pl.BlockSpec((1, _D), lambda: (0, 0)),
        out_shape=jax.ShapeDtypeStruct((1, _D), jnp.float32),
        scratch_shapes=[pltpu.VMEM((_K, _D), jnp.float32),
                        pltpu.SemaphoreType.DMA],
    )(logits, entity_embeddings, Wv, bv.reshape(1, _H), Wo, bo.reshape(1, _D))

    return out.reshape(_D)


# speedup vs baseline: 1.8841x; 1.5325x over previous
import jax
import jax.numpy as jnp
from jax import lax
from jax.experimental import pallas as pl
from jax.experimental.pallas import tpu as pltpu

_N = 8192
_D = 768
_H = 256
_K = 32
_NBLK = 4
_BLK = _N // _NBLK

_HI = lax.Precision.HIGHEST


def _fused_body(c_ref, wq_ref, bq_ref, wk_ref, wv_ref, bv_ref, wo_ref, bo_ref,
                e_ref, out_ref, buf0, buf1, logits_s, rows_ref,
                sem0, sem1, semg):
    bufs = [buf0, buf1]
    sems = [sem0, sem1]

    def stream_in(j):
        return pltpu.make_async_copy(
            e_ref.at[pl.ds(j * _BLK, _BLK)], bufs[j % 2], sems[j % 2])

    stream_in(0).start()
    stream_in(1).start()

    q = jnp.dot(c_ref[...], wq_ref[...], preferred_element_type=jnp.float32,
                precision=_HI) + bq_ref[...]
    w = lax.dot_general(q, wk_ref[...], (((1,), (1,)), ((), ())),
                        preferred_element_type=jnp.float32,
                        precision=_HI)

    for j in range(_NBLK):
        stream_in(j).wait()
        r = jnp.sum(bufs[j % 2][...] * w, axis=1) * 0.0625
        logits_s[j, :] = r
        if j + 2 < _NBLK:
            stream_in(j + 2).start()

    l = logits_s[...]
    m = jnp.max(l)
    p = jnp.exp(l - m)
    zinv = 1.0 / jnp.sum(p)
    flat = (lax.broadcasted_iota(jnp.int32, (_NBLK, _BLK), 0) * _BLK +
            lax.broadcasted_iota(jnp.int32, (_NBLK, _BLK), 1))
    big = jnp.int32(2 ** 30)

    work = p
    idxs = []
    svals = []
    for j in range(_K):
        pj = jnp.max(work)
        ij = jnp.min(jnp.where(work == pj, flat, big))
        idxs.append(ij)
        svals.append(pj * zinv)
        work = jnp.where(flat == ij, 0.0, work)

    for j in range(_K):
        pltpu.make_async_copy(e_ref.at[pl.ds(idxs[j], 1)],
                              rows_ref.at[pl.ds(j, 1)], semg).start()
    for j in range(_K):
        pltpu.make_async_copy(e_ref.at[pl.ds(idxs[j], 1)],
                              rows_ref.at[pl.ds(j, 1)], semg).wait()

    u = rows_ref[0, :][None, :] * svals[0]
    s_sum = svals[0]
    for j in range(1, _K):
        u = u + rows_ref[j, :][None, :] * svals[j]
        s_sum = s_sum + svals[j]

    hv = jnp.dot(u, wv_ref[...], preferred_element_type=jnp.float32,
                 precision=_HI) + s_sum * bv_ref[...]
    out = jnp.dot(hv, wo_ref[...], preferred_element_type=jnp.float32,
                  precision=_HI) + bo_ref[...]
    out_ref[...] = out


def kernel(class_embedding, entity_embeddings, Wq, bq, Wk, bk, Wv, bv, Wo, bo):
    del bk
    c2 = class_embedding.reshape(1, _D)

    vm = pl.BlockSpec(memory_space=pltpu.VMEM)
    out = pl.pallas_call(
        _fused_body,
        in_specs=[vm, vm, vm, vm, vm, vm, vm, vm,
                  pl.BlockSpec(memory_space=pl.ANY)],
        out_specs=vm,
        out_shape=jax.ShapeDtypeStruct((1, _D), jnp.float32),
        scratch_shapes=[
            pltpu.VMEM((_BLK, _D), jnp.float32),
            pltpu.VMEM((_BLK, _D), jnp.float32),
            pltpu.VMEM((_NBLK, _BLK), jnp.float32),
            pltpu.VMEM((_K, _D), jnp.float32),
            pltpu.SemaphoreType.DMA,
            pltpu.SemaphoreType.DMA,
            pltpu.SemaphoreType.DMA,
        ],
    )(c2, Wq, bq.reshape(1, _H), Wk, Wv, bv.reshape(1, _H), Wo,
      bo.reshape(1, _D), entity_embeddings)

    return out.reshape(_D)
